# baseline (speedup 1.0000x reference)
import functools

import jax
import jax.numpy as jnp
from jax import lax
from jax.experimental import pallas as pl
from jax.experimental.pallas import tpu as pltpu
from jax.experimental.pallas import tpu_sc as plsc

D = 1024
B = 4
S = 2048
NC = 2
NS = 16
NW = NC * NS
P_PER_W = S // NW
CHUNK = 8
NPAIR = 2
NCHUNK = P_PER_W // CHUNK
NU = NCHUNK * NPAIR
NB = 6
NPB = 3
LOOKA = 2
LANES = 16
SCALE = 32.0


def _embed_body(tok_hbm, pe_hbm, table_hbm, out_hbm,
                idx_v, rows0, rows1, rows2, rows3, rows4, rows5,
                pe0, pe1, pe2, i_sem, g_sem, p_sem, s_sem):
    c = lax.axis_index("c")
    s = lax.axis_index("s")
    wid = s * NC + c
    p0 = wid * P_PER_W

    icps = [
        pltpu.async_copy(tok_hbm.at[b, pl.ds(p0, P_PER_W)],
                         idx_v.at[b], i_sem)
        for b in range(B)
    ]
    for cp in icps:
        cp.wait()

    rows_bufs = (rows0, rows1, rows2, rows3, rows4, rows5)
    pe_bufs = (pe0, pe1, pe2)

    def gather(u, buf):
        k, p = u // NPAIR, u % NPAIR
        return [
            pltpu.async_copy(
                table_hbm.at[idx_v.at[2 * p + i, pl.ds(k * CHUNK, CHUNK)]],
                buf.at[pl.ds(i * CHUNK, CHUNK)], g_sem)
            for i in range(2)
        ]

    def pe_load(k, buf):
        src = pe_hbm.at[pl.ds(p0 + k * CHUNK, CHUNK)]
        return pltpu.async_copy(src, buf, p_sem)

    gathers = [None] * NU
    pe_loads = [None] * NCHUNK
    scatters = [None] * NU

    for u in range(LOOKA):
        gathers[u] = gather(u, rows_bufs[u % NB])
        if u % NPAIR == 0:
            pe_loads[u // NPAIR] = pe_load(u // NPAIR, pe_bufs[(u // NPAIR) % NPB])

    for u in range(NU):
        k, p = u // NPAIR, u % NPAIR
        ua = u + LOOKA
        if ua < NU:
            if ua - NB >= 0:
                for cp in scatters[ua - NB]:
                    cp.wait()
            gathers[ua] = gather(ua, rows_bufs[ua % NB])
            if ua % NPAIR == 0:
                ka = ua // NPAIR
                pe_loads[ka] = pe_load(ka, pe_bufs[ka % NPB])
        for cp in gathers[u]:
            cp.wait()
        if p == 0:
            pe_loads[k].wait()

        rows = rows_bufs[u % NB]
        peb = pe_bufs[k % NPB]

        def jbody(j, carry, rows=rows, peb=peb):
            sl = pl.ds(j * LANES, LANES)
            for r in range(CHUNK):
                pv = peb[r, sl]
                for i in range(2):
                    row = i * CHUNK + r
                    rows[row, sl] = rows[row, sl] * SCALE + pv
            return carry

        lax.fori_loop(0, D // LANES, jbody, 0)

        scatters[u] = [
            pltpu.async_copy(
                rows.at[pl.ds(i * CHUNK, CHUNK)],
                out_hbm.at[pl.ds((2 * p + i) * S + p0 + k * CHUNK, CHUNK)],
                s_sem)
            for i in range(2)
        ]

    for u in range(max(0, NU - NB), NU):
        for cp in scatters[u]:
            cp.wait()


def kernel(tokens, table, pe):
    mesh = plsc.VectorSubcoreMesh(core_axis_name="c", subcore_axis_name="s")
    run = functools.partial(
        pl.kernel,
        mesh=mesh,
        out_type=jax.ShapeDtypeStruct((B * S, D), jnp.float32),
        scratch_types=[
            pltpu.VMEM((B, P_PER_W), jnp.int32),
        ] + [pltpu.VMEM((2 * CHUNK, D), jnp.float32) for _ in range(NB)]
          + [pltpu.VMEM((CHUNK, D), jnp.float32) for _ in range(NPB)]
          + [
            pltpu.SemaphoreType.DMA,
            pltpu.SemaphoreType.DMA,
            pltpu.SemaphoreType.DMA,
            pltpu.SemaphoreType.DMA,
        ],
    )(_embed_body)
    out = run(tokens.astype(jnp.int32), pe, table)
    return out.reshape(B, S, D)
